# baseline (speedup 1.0000x reference)
import functools

import jax
import jax.numpy as jnp
from jax import lax
from jax.experimental import pallas as pl
from jax.experimental.pallas import tpu as pltpu
from jax.experimental.pallas import tpu_sc as plsc

_D = 64
_K = 1024
_N = 16384
_COMMIT = 0.25

_BM = 1024
_HB = 2
_NP = 2
_NROWS = _N // _NP
_NB = _NROWS // (_BM * _HB)

_NC = 2
_NS = 16
_NW = _NC * _NS
_BPW = _NROWS // _NW
_CHUNK = 128
_NCHUNK = _BPW // _CHUNK


def _half_argmin(zt, cbt, c2, ii):
    z2 = jnp.sum(zt * zt, axis=0)[:, None]
    ndot = lax.dot_general(zt * -2.0, cbt, (((0,), (0,)), ((), ())),
                           preferred_element_type=jnp.float32,
                           precision=lax.Precision.DEFAULT)
    d = (z2 + ndot) + c2[None, :]
    m = jnp.min(d, axis=1, keepdims=True)
    idx = jnp.min(jnp.where(d == m, ii, float(_K)), axis=1).astype(jnp.int32)
    return m, idx.reshape(_BM // 128, 128)


def _tc_distance_argmin(z_ref, cb_ref, io_ref, idx_ref, com_ref, emb_ref,
                        acc_ref):
    i = pl.program_id(0)
    cbt = cb_ref[...]
    c2 = jnp.sum(cbt * cbt, axis=0)
    ii = io_ref[...]
    msum = 0.0
    for h in range(_HB):
        m, idxh = _half_argmin(z_ref[h], cbt, c2, ii)
        idx_ref[pl.ds(h * (_BM // 128), _BM // 128), :] = idxh
        msum += jnp.sum(m)

    @pl.when(i == 0)
    def _init():
        acc_ref[0] = 0.0

    acc_ref[0] += msum

    @pl.when(i == pl.num_programs(0) - 1)
    def _fin():
        mse = acc_ref[0] / float(_N * _D)
        com_ref[0, 0] = _COMMIT * mse
        emb_ref[0, 0] = mse


def _tc_call(z_t, cb_t):
    return pl.pallas_call(
        _tc_distance_argmin,
        grid=(_NB,),
        in_specs=[
            pl.BlockSpec((_HB, _D, _BM), lambda i: (i, 0, 0)),
            pl.BlockSpec((_D, _K), lambda i: (0, 0)),
            pl.BlockSpec((1, _K), lambda i: (0, 0)),
        ],
        out_specs=[
            pl.BlockSpec((_HB * _BM // 128, 128), lambda i: (i, 0)),
            pl.BlockSpec(memory_space=pltpu.SMEM),
            pl.BlockSpec(memory_space=pltpu.SMEM),
        ],
        out_shape=[
            jax.ShapeDtypeStruct((_NROWS // 128, 128), jnp.int32),
            jax.ShapeDtypeStruct((1, 1), jnp.float32),
            jax.ShapeDtypeStruct((1, 1), jnp.float32),
        ],
        scratch_shapes=[pltpu.SMEM((1,), jnp.float32)],
        compiler_params=pltpu.CompilerParams(
            dimension_semantics=("arbitrary",),
        ),
    )(z_t, cb_t, jnp.arange(_K, dtype=jnp.float32)[None, :])


def _sc_gather_body(cb_hbm, idx_hbm, out_hbm, idx_v, rows_v, sem):
    wid = lax.axis_index("s") * _NC + lax.axis_index("c")
    base = wid * _BPW
    pltpu.sync_copy(idx_hbm.at[pl.ds(base, _BPW)], idx_v)
    copies = []
    for j in range(_NCHUNK):
        copies.append(pltpu.async_copy(
            cb_hbm.at[idx_v.at[pl.ds(j * _CHUNK, _CHUNK)]],
            rows_v.at[pl.ds(j * _CHUNK, _CHUNK)],
            sem))
    for c in copies:
        c.wait()
    pltpu.sync_copy(rows_v, out_hbm.at[pl.ds(base, _BPW)])


@functools.cache
def _sc_gather_call():
    return pl.kernel(
        _sc_gather_body,
        mesh=plsc.VectorSubcoreMesh(core_axis_name="c", subcore_axis_name="s"),
        out_type=jax.ShapeDtypeStruct((_NROWS, 128), jnp.float32),
        scratch_types=[
            pltpu.VMEM((_BPW,), jnp.int32),
            pltpu.VMEM((_BPW, 128), jnp.float32),
            pltpu.SemaphoreType.DMA,
        ],
        compiler_params=pltpu.CompilerParams(use_tc_tiling_on_sc=False),
    )


def kernel(z_e, codebook):
    cb = jnp.asarray(codebook, dtype=jnp.float32)
    z_t = lax.transpose(z_e, (0, 2, 1))
    cb_t = lax.transpose(cb, (1, 0))
    cb_pad = jnp.pad(cb, ((0, 0), (0, 128 - _D)))
    nb = 16 // _NP
    idxs, qs, coms, embs = [], [], [], []
    for p in range(_NP):
        idx2, com, emb = _tc_call(
            lax.slice_in_dim(z_t, p * nb, (p + 1) * nb, axis=0), cb_t)
        idx = jnp.reshape(idx2, (_NROWS,))
        qs.append(_sc_gather_call()(cb_pad, idx))
        idxs.append(idx)
        coms.append(com[0, 0])
        embs.append(emb[0, 0])
    q = jnp.concatenate(qs, axis=0)
    quantized = jnp.reshape(q[:, :_D], z_e.shape)
    return (quantized, sum(coms), sum(embs), jnp.concatenate(idxs))


# speedup vs baseline: 1.1725x; 1.1725x over previous
import functools

import jax
import jax.numpy as jnp
from jax import lax
from jax.experimental import pallas as pl
from jax.experimental.pallas import tpu as pltpu
from jax.experimental.pallas import tpu_sc as plsc

_D = 64
_K = 1024
_N = 16384
_COMMIT = 0.25

_BM = 1024
_HB = 2
_NB = _N // (_BM * _HB)

_NC = 2
_NS = 16
_NW = _NC * _NS
_BPW = _N // _NW
_CHUNK = 128
_NCHUNK = _BPW // _CHUNK


def _half_argmin(zt, cbt, c2, ii):
    z2 = jnp.sum(zt * zt, axis=0)[:, None]
    ndot = lax.dot_general(zt * -2.0, cbt, (((0,), (0,)), ((), ())),
                           preferred_element_type=jnp.float32,
                           precision=lax.Precision.DEFAULT)
    d = (z2 + ndot) + c2[None, :]
    m = jnp.min(d, axis=1, keepdims=True)
    idx = jnp.min(jnp.where(d == m, ii, float(_K)), axis=1).astype(jnp.int32)
    return m, idx.reshape(_BM // 128, 128)


def _tc_distance_argmin(z_ref, cb_ref, io_ref, idx_ref, com_ref, emb_ref,
                        acc_ref):
    i = pl.program_id(0)
    cbt = cb_ref[...]
    c2 = jnp.sum(cbt * cbt, axis=0)
    ii = io_ref[...]
    msum = 0.0
    for h in range(_HB):
        m, idxh = _half_argmin(z_ref[h], cbt, c2, ii)
        idx_ref[pl.ds(h * (_BM // 128), _BM // 128), :] = idxh
        msum += jnp.sum(m)

    @pl.when(i == 0)
    def _init():
        acc_ref[0] = 0.0

    acc_ref[0] += msum

    @pl.when(i == pl.num_programs(0) - 1)
    def _fin():
        mse = acc_ref[0] / float(_N * _D)
        com_ref[0, 0] = _COMMIT * mse
        emb_ref[0, 0] = mse


def _tc_call(z_t, cb_t):
    return pl.pallas_call(
        _tc_distance_argmin,
        grid=(_NB,),
        in_specs=[
            pl.BlockSpec((_HB, _D, _BM), lambda i: (i, 0, 0)),
            pl.BlockSpec((_D, _K), lambda i: (0, 0)),
            pl.BlockSpec((1, _K), lambda i: (0, 0)),
        ],
        out_specs=[
            pl.BlockSpec((_HB * _BM // 128, 128), lambda i: (i, 0)),
            pl.BlockSpec(memory_space=pltpu.SMEM),
            pl.BlockSpec(memory_space=pltpu.SMEM),
        ],
        out_shape=[
            jax.ShapeDtypeStruct((_N // 128, 128), jnp.int32),
            jax.ShapeDtypeStruct((1, 1), jnp.float32),
            jax.ShapeDtypeStruct((1, 1), jnp.float32),
        ],
        scratch_shapes=[pltpu.SMEM((1,), jnp.float32)],
        compiler_params=pltpu.CompilerParams(
            dimension_semantics=("arbitrary",),
        ),
    )(z_t, cb_t, jnp.arange(_K, dtype=jnp.float32)[None, :])


def _sc_gather_body(cb_hbm, idx_hbm, out_hbm, idx_v, rows_v, sem):
    wid = lax.axis_index("s") * _NC + lax.axis_index("c")
    base = wid * _BPW
    pltpu.sync_copy(idx_hbm.at[pl.ds(base, _BPW)], idx_v)
    copies = []
    for j in range(_NCHUNK):
        copies.append(pltpu.async_copy(
            cb_hbm.at[idx_v.at[pl.ds(j * _CHUNK, _CHUNK)]],
            rows_v.at[pl.ds(j * _CHUNK, _CHUNK)],
            sem))
    for c in copies:
        c.wait()
    pltpu.sync_copy(rows_v, out_hbm.at[pl.ds(base, _BPW)])


@functools.cache
def _sc_gather_call():
    return pl.kernel(
        _sc_gather_body,
        mesh=plsc.VectorSubcoreMesh(core_axis_name="c", subcore_axis_name="s"),
        out_type=jax.ShapeDtypeStruct((_N, 128), jnp.float32),
        scratch_types=[
            pltpu.VMEM((_BPW,), jnp.int32),
            pltpu.VMEM((_BPW, 128), jnp.float32),
            pltpu.SemaphoreType.DMA,
        ],
        compiler_params=pltpu.CompilerParams(use_tc_tiling_on_sc=False),
    )


def kernel(z_e, codebook):
    cb = jnp.asarray(codebook, dtype=jnp.float32)
    z_t = lax.transpose(z_e, (0, 2, 1))
    cb_t = lax.transpose(cb, (1, 0))
    idx2, com, emb = _tc_call(z_t, cb_t)
    idx = jnp.reshape(idx2, (_N,))
    cb_pad = jnp.pad(cb, ((0, 0), (0, 128 - _D)))
    q = _sc_gather_call()(cb_pad, idx)
    quantized = jnp.reshape(q[:, :_D], z_e.shape)
    return quantized, com[0, 0], emb[0, 0], idx


# speedup vs baseline: 1.2881x; 1.0986x over previous
import functools

import jax
import jax.numpy as jnp
from jax import lax
from jax.experimental import pallas as pl
from jax.experimental.pallas import tpu as pltpu
from jax.experimental.pallas import tpu_sc as plsc

_D = 64
_K = 1024
_N = 16384
_COMMIT = 0.25

_BM = 1024
_HB = 2
_NB = _N // (_BM * _HB)

_NC = 2
_NS = 16
_NW = _NC * _NS
_BPW = _N // _NW
_CHUNK = 128
_NCHUNK = _BPW // _CHUNK


def _half_argmin(zt, cbt, c2, ii):
    z2 = jnp.sum(zt * zt, axis=0)[:, None]
    ndot = lax.dot_general(zt * -2.0, cbt, (((0,), (0,)), ((), ())),
                           preferred_element_type=jnp.float32,
                           precision=lax.Precision.DEFAULT)
    d = (z2 + ndot) + c2[None, :]
    m = jnp.min(d, axis=1, keepdims=True)
    idx = jnp.min(jnp.where(d == m, ii, float(_K)), axis=1).astype(jnp.int32)
    return m, idx.reshape(_BM // 128, 128)


def _tc_distance_argmin(z_ref, cb_ref, io_ref, idx_ref, com_ref, emb_ref,
                        acc_ref):
    i = pl.program_id(0)
    cbt = cb_ref[...]
    c2 = jnp.sum(cbt * cbt, axis=0)
    ii = io_ref[...]
    msum = 0.0
    for h in range(_HB):
        m, idxh = _half_argmin(z_ref[h], cbt, c2, ii)
        idx_ref[pl.ds(h * (_BM // 128), _BM // 128), :] = idxh
        msum += jnp.sum(m)

    @pl.when(i == 0)
    def _init():
        acc_ref[0] = 0.0

    acc_ref[0] += msum

    @pl.when(i == pl.num_programs(0) - 1)
    def _fin():
        mse = acc_ref[0] / float(_N * _D)
        com_ref[0, 0] = _COMMIT * mse
        emb_ref[0, 0] = mse


def _tc_call(z_t, cb_t):
    return pl.pallas_call(
        _tc_distance_argmin,
        grid=(_NB,),
        in_specs=[
            pl.BlockSpec((_HB, _D, _BM), lambda i: (i, 0, 0)),
            pl.BlockSpec((_D, _K), lambda i: (0, 0)),
            pl.BlockSpec((1, _K), lambda i: (0, 0)),
        ],
        out_specs=[
            pl.BlockSpec((_HB * _BM // 128, 128), lambda i: (i, 0)),
            pl.BlockSpec(memory_space=pltpu.SMEM),
            pl.BlockSpec(memory_space=pltpu.SMEM),
        ],
        out_shape=[
            jax.ShapeDtypeStruct((_N // 128, 128), jnp.int32),
            jax.ShapeDtypeStruct((1, 1), jnp.float32),
            jax.ShapeDtypeStruct((1, 1), jnp.float32),
        ],
        scratch_shapes=[pltpu.SMEM((1,), jnp.float32)],
        compiler_params=pltpu.CompilerParams(
            dimension_semantics=("arbitrary",),
        ),
    )(z_t, cb_t, jnp.arange(_K, dtype=jnp.float32)[None, :])


def _sc_gather_body(cb_hbm, idx_hbm, out_hbm, idx_v, rows_v, sem):
    wid = lax.axis_index("s") * _NC + lax.axis_index("c")
    base = wid * _BPW
    pltpu.sync_copy(idx_hbm.at[pl.ds(base, _BPW)], idx_v)
    copies = []
    for j in range(_NCHUNK):
        copies.append(pltpu.async_copy(
            cb_hbm.at[idx_v.at[pl.ds(j * _CHUNK, _CHUNK)]],
            rows_v.at[pl.ds(j * _CHUNK, _CHUNK)],
            sem))
    for c in copies:
        c.wait()
    pltpu.sync_copy(rows_v, out_hbm.at[pl.ds(base, _BPW), pl.ds(0, _D)])


@functools.cache
def _sc_gather_call():
    return pl.kernel(
        _sc_gather_body,
        mesh=plsc.VectorSubcoreMesh(core_axis_name="c", subcore_axis_name="s"),
        out_type=jax.ShapeDtypeStruct((_N, 128), jnp.float32),
        scratch_types=[
            pltpu.VMEM((_BPW,), jnp.int32),
            pltpu.VMEM((_BPW, _D), jnp.float32),
            pltpu.SemaphoreType.DMA,
        ],
        compiler_params=pltpu.CompilerParams(use_tc_tiling_on_sc=False),
    )


def kernel(z_e, codebook):
    cb = jnp.asarray(codebook, dtype=jnp.float32)
    z_t = lax.transpose(z_e, (0, 2, 1))
    cb_t = lax.transpose(cb, (1, 0))
    idx2, com, emb = _tc_call(z_t, cb_t)
    idx = jnp.reshape(idx2, (_N,))
    q = _sc_gather_call()(cb, idx)
    quantized = jnp.reshape(q[:, :_D], z_e.shape)
    return quantized, com[0, 0], emb[0, 0], idx
